# baseline (speedup 1.0000x reference)
import functools

import jax
import jax.numpy as jnp
from jax import lax
from jax.experimental import pallas as pl
from jax.experimental.pallas import tpu as pltpu
from jax.experimental.pallas import tpu_sc as plsc

N = 10000
NNZ = 160000
NE = 2000
NFEAT = 256
HID = 256
NCLASS = 40
NCLASS_PAD = 256

NS = 16
NC = 2
W = 80
PER_TILE = NNZ // NS
NWIN = PER_TILE // W
IB = 25
NIB = NWIN // IB
NEP = 2048
NVP = 10240
SB = 32

WM = 50
TM = 40
NBLKM = PER_TILE // (WM * TM)

_MESH = plsc.VectorSubcoreMesh(core_axis_name="c", subcore_axis_name="s")
_f32 = jnp.float32
_i32 = jnp.int32


def _zero_2d(buf, rows, cols):
    z = jnp.zeros((16,), _f32)

    @pl.loop(0, rows)
    def _(r):
        for j in range(cols // 16):
            buf[r, pl.ds(j * 16, 16)] = z


def _counts_body(V4, E4, inv_e_hbm, inv_v_hbm, vblk, eblk, ones, cbuf,
                 cnt_e_sh, cnt_v_sh):
    c = lax.axis_index("c")
    s = lax.axis_index("s")
    ne_t = NEP // NS
    nv_t = NVP // NS

    for j in range(W // 16):
        ones[pl.ds(j * 16, 16)] = jnp.ones((16,), _f32)
    for j in range(nv_t // 16):
        cbuf[pl.ds(j * 16, 16)] = jnp.zeros((16,), _f32)
    pltpu.sync_copy(cbuf, cnt_v_sh.at[pl.ds(s * nv_t, nv_t)])
    pltpu.sync_copy(cbuf.at[pl.ds(0, ne_t)], cnt_e_sh.at[pl.ds(s * ne_t, ne_t)])
    plsc.subcore_barrier()

    @pl.loop(0, NIB)
    def _(bk):
        pltpu.sync_copy(V4.at[s, bk], vblk)
        pltpu.sync_copy(E4.at[s, bk], eblk)

        @pl.loop(0, IB)
        def _(wi):
            pltpu.sync_copy(ones, cnt_e_sh.at[eblk.at[wi]], add=True)
            pltpu.sync_copy(ones, cnt_v_sh.at[vblk.at[wi]], add=True)

    plsc.subcore_barrier()

    pltpu.sync_copy(cnt_v_sh.at[pl.ds(s * nv_t, nv_t)], cbuf)

    @pl.loop(0, nv_t // 16)
    def _(j):
        x = cbuf[pl.ds(j * 16, 16)]
        cbuf[pl.ds(j * 16, 16)] = 1.0 / jnp.maximum(x, 1.0)

    @pl.when(c == 0)
    def _():
        pltpu.sync_copy(cbuf, inv_v_hbm.at[pl.ds(s * nv_t, nv_t)])

    pltpu.sync_copy(cnt_e_sh.at[pl.ds(s * ne_t, ne_t)], cbuf.at[pl.ds(0, ne_t)])

    @pl.loop(0, ne_t // 16)
    def _(j):
        x = cbuf[pl.ds(j * 16, 16)]
        cbuf[pl.ds(j * 16, 16)] = 1.0 / jnp.maximum(x, 1.0)

    @pl.when(c == 0)
    def _():
        pltpu.sync_copy(cbuf.at[pl.ds(0, ne_t)], inv_e_hbm.at[pl.ds(s * ne_t, ne_t)])


_counts_kernel = pl.kernel(
    _counts_body,
    out_type=(jax.ShapeDtypeStruct((NEP,), _f32),
              jax.ShapeDtypeStruct((NVP,), _f32)),
    mesh=_MESH,
    scratch_types=[
        pltpu.VMEM((IB, W), _i32),
        pltpu.VMEM((IB, W), _i32),
        pltpu.VMEM((W,), _f32),
        pltpu.VMEM((NVP // NS,), _f32),
        pltpu.VMEM_SHARED((NEP,), _f32),
        pltpu.VMEM_SHARED((NVP,), _f32),
    ],
)


def _pipelined_stage(c, s, V4, E4, vblk, eblk, rows2, sem_g, sem_s,
                     gather_fire, gather_wait, scat_ref):
    @pl.loop(0, NBLKM)
    def _(bk):
        pltpu.sync_copy(V4.at[s, bk], vblk)
        pltpu.sync_copy(E4.at[s, bk], eblk)
        gather_fire(0, rows2.at[0])

        @pl.loop(0, TM)
        def _(wi):
            b = wi % 2
            gather_wait(rows2.at[b])

            @pl.when(wi + 1 < TM)
            def _():
                @pl.when(wi >= 1)
                def _():
                    pltpu.make_async_copy(
                        rows2.at[1 - b], scat_ref.at[eblk.at[wi]], sem_s).wait()

                gather_fire(wi + 1, rows2.at[1 - b])

            pltpu.async_copy(rows2.at[b], scat_ref.at[eblk.at[wi]], sem_s,
                             add=True)

        for _k in range(2):
            pltpu.make_async_copy(
                rows2.at[_k], scat_ref.at[eblk.at[0]], sem_s).wait()


def _msgpass_body(F2, xwl, xwr, V4, E4, inv_e_hbm,
                  outl, outr, efl, efr, vblk, eblk, rows2, invebuf,
                  sem_g, sem_s, e_sh, v_sh):
    c = lax.axis_index("c")
    s = lax.axis_index("s")
    ne_blks = NEP // NS // SB
    nv_blks = NVP // NS // SB
    zrows = rows2.at[0, pl.ds(0, SB), :]
    nref = rows2.at[0]

    pltpu.sync_copy(inv_e_hbm.at[pl.ds(s * (NEP // NS), NEP // NS)], invebuf)
    _zero_2d(nref, SB, F2)
    for k in range(ne_blks):
        pltpu.sync_copy(zrows, e_sh.at[pl.ds((s * ne_blks + k) * SB, SB), :])
    for k in range(nv_blks):
        pltpu.sync_copy(zrows, v_sh.at[pl.ds((s * nv_blks + k) * SB, SB), :])
    plsc.subcore_barrier()

    def fire1(wi, dst):
        @pl.when(c == 0)
        def _():
            pltpu.async_copy(xwl.at[vblk.at[wi]], dst, sem_g)

        @pl.when(c == 1)
        def _():
            pltpu.async_copy(xwr.at[vblk.at[wi]], dst, sem_g)

    def wait1(dst):
        pltpu.make_async_copy(xwl.at[vblk.at[0]], dst, sem_g).wait()

    _pipelined_stage(c, s, V4, E4, vblk, eblk, rows2, sem_g, sem_s,
                     fire1, wait1, e_sh)
    plsc.subcore_barrier()

    for k in range(ne_blks):
        base = (s * ne_blks + k) * SB
        lbase = k * SB
        pltpu.sync_copy(e_sh.at[pl.ds(base, SB), :], zrows)

        @pl.loop(0, SB // 16)
        def _(g):
            ivgrp = invebuf[pl.ds(lbase + g * 16, 16)]
            for r16 in range(16):
                iv = jnp.broadcast_to(ivgrp[r16], (16,))
                for j in range(F2 // 16):
                    r = g * 16 + r16
                    nref[r, pl.ds(j * 16, 16)] = nref[r, pl.ds(j * 16, 16)] * iv

        @pl.when(c == 0)
        def _():
            pltpu.sync_copy(zrows, efl.at[pl.ds(base, SB), :])

        @pl.when(c == 1)
        def _():
            pltpu.sync_copy(zrows, efr.at[pl.ds(base, SB), :])

    plsc.subcore_barrier()

    def fire2(wi, dst):
        @pl.when(c == 0)
        def _():
            pltpu.async_copy(efl.at[eblk.at[wi]], dst, sem_g)

        @pl.when(c == 1)
        def _():
            pltpu.async_copy(efr.at[eblk.at[wi]], dst, sem_g)

    def wait2(dst):
        pltpu.make_async_copy(efl.at[eblk.at[0]], dst, sem_g).wait()

    _pipelined_stage(c, s, E4, V4, eblk, vblk, rows2, sem_g, sem_s,
                     fire2, wait2, v_sh)
    plsc.subcore_barrier()

    nv_t = NVP // NS

    @pl.when(c == 0)
    def _():
        pltpu.sync_copy(v_sh.at[pl.ds(s * nv_t, nv_t), :],
                        outl.at[pl.ds(s * nv_t, nv_t), :])

    @pl.when(c == 1)
    def _():
        pltpu.sync_copy(v_sh.at[pl.ds(s * nv_t, nv_t), :],
                        outr.at[pl.ds(s * nv_t, nv_t), :])


@functools.cache
def _msgpass_kernel(F2):
    return pl.kernel(
        functools.partial(_msgpass_body, F2),
        out_type=(jax.ShapeDtypeStruct((NVP, F2), _f32),
                  jax.ShapeDtypeStruct((NVP, F2), _f32),
                  jax.ShapeDtypeStruct((NEP, F2), _f32),
                  jax.ShapeDtypeStruct((NEP, F2), _f32)),
        mesh=_MESH,
        scratch_types=[
            pltpu.VMEM((TM, WM), _i32),
            pltpu.VMEM((TM, WM), _i32),
            pltpu.VMEM((2, WM, F2), _f32),
            pltpu.VMEM((NEP // NS,), _f32),
            pltpu.SemaphoreType.DMA,
            pltpu.SemaphoreType.DMA,
            pltpu.VMEM_SHARED((NEP, F2), _f32),
            pltpu.VMEM_SHARED((NVP, F2), _f32),
        ],
    )


_RBLK = 1000


def _matmul2_body(split, relu, wa_ref, wb_ref, b_ref, xa_ref, xb_ref, d_ref,
                  ol_ref, or_ref):
    xa, xb = xa_ref[...], xb_ref[...]
    if relu:
        xa, xb = jnp.maximum(xa, 0.0), jnp.maximum(xb, 0.0)
    acc = jnp.dot(xa, wa_ref[...], preferred_element_type=_f32)
    acc += jnp.dot(xb, wb_ref[...], preferred_element_type=_f32)
    acc = acc * d_ref[...] + b_ref[...]
    ol_ref[...] = acc[:, :split]
    or_ref[...] = acc[:, split:]


def _matmul2(xa, xb, w, b, d=None, relu=False):
    if d is None:
        d = jnp.ones((N, 1), _f32)
    fin_a = xa.shape[1]
    fout = w.shape[1]
    split = fout // 2
    grid = (N // _RBLK,)
    return pl.pallas_call(
        functools.partial(_matmul2_body, split, relu),
        grid=grid,
        in_specs=[
            pl.BlockSpec((fin_a, fout), lambda i: (0, 0)),
            pl.BlockSpec((xb.shape[1], fout), lambda i: (0, 0)),
            pl.BlockSpec((1, fout), lambda i: (0, 0)),
            pl.BlockSpec((_RBLK, fin_a), lambda i: (i, 0)),
            pl.BlockSpec((_RBLK, xb.shape[1]), lambda i: (i, 0)),
            pl.BlockSpec((_RBLK, 1), lambda i: (i, 0)),
        ],
        out_specs=[
            pl.BlockSpec((_RBLK, split), lambda i: (i, 0)),
            pl.BlockSpec((_RBLK, split), lambda i: (i, 0)),
        ],
        out_shape=[
            jax.ShapeDtypeStruct((N, split), _f32),
            jax.ShapeDtypeStruct((N, split), _f32),
        ],
    )(w[:fin_a], w[fin_a:], b.reshape(1, -1), xa, xb, d)


def _logsoftmax_body(xl_ref, xr_ref, d_ref, o_ref):
    x = jnp.concatenate([xl_ref[...], xr_ref[...]], axis=1) * d_ref[...]
    mask = lax.broadcasted_iota(_i32, x.shape, 1) < NCLASS
    neg = jnp.full_like(x, -jnp.inf)
    z = jnp.where(mask, x, neg)
    m = jnp.max(z, axis=1, keepdims=True)
    ex = jnp.where(mask, jnp.exp(x - m), 0.0)
    ssum = jnp.sum(ex, axis=1, keepdims=True)
    o_ref[...] = x - m - jnp.log(ssum)


def _logsoftmax(xl, xr, d):
    half = xl.shape[1]
    grid = (N // _RBLK,)
    return pl.pallas_call(
        _logsoftmax_body,
        grid=grid,
        in_specs=[
            pl.BlockSpec((_RBLK, half), lambda i: (i, 0)),
            pl.BlockSpec((_RBLK, half), lambda i: (i, 0)),
            pl.BlockSpec((_RBLK, 1), lambda i: (i, 0)),
        ],
        out_specs=pl.BlockSpec((_RBLK, 2 * half), lambda i: (i, 0)),
        out_shape=jax.ShapeDtypeStruct((N, 2 * half), _f32),
    )(xl, xr, d)


def kernel(X, V, E, W1, b1, W2, b2, W3, b3):
    V3 = V.astype(_i32).reshape(NS, NIB, IB, W)
    E3 = E.astype(_i32).reshape(NS, NIB, IB, W)
    V4 = V.astype(_i32).reshape(NS, NBLKM, TM, WM)
    E4 = E.astype(_i32).reshape(NS, NBLKM, TM, WM)
    inv_e, inv_v = _counts_kernel(V3, E3)
    d = inv_v[:N].reshape(N, 1)

    xl, xr = _matmul2(X[:, :NFEAT // 2], X[:, NFEAT // 2:], W1, b1)
    hl, hr, _, _ = _msgpass_kernel(HID // 2)(xl, xr, V4, E4, inv_e)

    xl, xr = _matmul2(hl[:N], hr[:N], W2, b2, d=d, relu=True)
    hl, hr, _, _ = _msgpass_kernel(HID // 2)(xl, xr, V4, E4, inv_e)

    w3p = jnp.pad(W3, ((0, 0), (0, NCLASS_PAD - NCLASS)))
    b3p = jnp.pad(b3, (0, NCLASS_PAD - NCLASS))
    xl, xr = _matmul2(hl[:N], hr[:N], w3p, b3p, d=d, relu=True)
    fl, fr, _, _ = _msgpass_kernel(NCLASS_PAD // 2)(xl, xr, V4, E4, inv_e)

    out = _logsoftmax(fl[:N], fr[:N], d)
    return out[:, :NCLASS]


# speedup vs baseline: 1.4168x; 1.4168x over previous
import functools

import jax
import jax.numpy as jnp
from jax import lax
from jax.experimental import pallas as pl
from jax.experimental.pallas import tpu as pltpu
from jax.experimental.pallas import tpu_sc as plsc

N = 10000
NNZ = 160000
NE = 2000
NFEAT = 256
HID = 256
NCLASS = 40
NCLASS_PAD = 256

NS = 16
NC = 2
W = 80
PER_TILE = NNZ // NS
NWIN = PER_TILE // W
IB = 25
NIB = NWIN // IB
NEP = 2048
NVP = 10240
SB = 32

WM = 40
TM = 50
NBLKM = PER_TILE // (WM * TM)
NBUF = 3

_MESH = plsc.VectorSubcoreMesh(core_axis_name="c", subcore_axis_name="s")
_f32 = jnp.float32
_i32 = jnp.int32


def _zero_2d(buf, rows, cols):
    z = jnp.zeros((16,), _f32)

    @pl.loop(0, rows)
    def _(r):
        for j in range(cols // 16):
            buf[r, pl.ds(j * 16, 16)] = z


def _counts_body(V4, E4, inv_e_hbm, inv_v_hbm, vblk, eblk, ones, cbuf,
                 cnt_e_sh, cnt_v_sh):
    c = lax.axis_index("c")
    s = lax.axis_index("s")
    ne_t = NEP // NS
    nv_t = NVP // NS

    for j in range(W // 16):
        ones[pl.ds(j * 16, 16)] = jnp.ones((16,), _f32)
    for j in range(nv_t // 16):
        cbuf[pl.ds(j * 16, 16)] = jnp.zeros((16,), _f32)
    pltpu.sync_copy(cbuf, cnt_v_sh.at[pl.ds(s * nv_t, nv_t)])
    pltpu.sync_copy(cbuf.at[pl.ds(0, ne_t)], cnt_e_sh.at[pl.ds(s * ne_t, ne_t)])
    plsc.subcore_barrier()

    @pl.loop(0, NIB)
    def _(bk):
        pltpu.sync_copy(V4.at[s, bk], vblk)
        pltpu.sync_copy(E4.at[s, bk], eblk)

        @pl.loop(0, IB)
        def _(wi):
            pltpu.sync_copy(ones, cnt_e_sh.at[eblk.at[wi]], add=True)
            pltpu.sync_copy(ones, cnt_v_sh.at[vblk.at[wi]], add=True)

    plsc.subcore_barrier()

    pltpu.sync_copy(cnt_v_sh.at[pl.ds(s * nv_t, nv_t)], cbuf)

    @pl.loop(0, nv_t // 16)
    def _(j):
        x = cbuf[pl.ds(j * 16, 16)]
        cbuf[pl.ds(j * 16, 16)] = 1.0 / jnp.maximum(x, 1.0)

    @pl.when(c == 0)
    def _():
        pltpu.sync_copy(cbuf, inv_v_hbm.at[pl.ds(s * nv_t, nv_t)])

    pltpu.sync_copy(cnt_e_sh.at[pl.ds(s * ne_t, ne_t)], cbuf.at[pl.ds(0, ne_t)])

    @pl.loop(0, ne_t // 16)
    def _(j):
        x = cbuf[pl.ds(j * 16, 16)]
        cbuf[pl.ds(j * 16, 16)] = 1.0 / jnp.maximum(x, 1.0)

    @pl.when(c == 0)
    def _():
        pltpu.sync_copy(cbuf.at[pl.ds(0, ne_t)], inv_e_hbm.at[pl.ds(s * ne_t, ne_t)])


_counts_kernel = pl.kernel(
    _counts_body,
    out_type=(jax.ShapeDtypeStruct((NEP,), _f32),
              jax.ShapeDtypeStruct((NVP,), _f32)),
    mesh=_MESH,
    scratch_types=[
        pltpu.VMEM((IB, W), _i32),
        pltpu.VMEM((IB, W), _i32),
        pltpu.VMEM((W,), _f32),
        pltpu.VMEM((NVP // NS,), _f32),
        pltpu.VMEM_SHARED((NEP,), _f32),
        pltpu.VMEM_SHARED((NVP,), _f32),
    ],
)


def _pipelined_stage(c, s, V4, E4, vblk, eblk, rows2, sem_g, sem_s,
                     gather_fire, gather_wait, scat_ref):
    @pl.loop(0, NBLKM)
    def _(bk):
        pltpu.sync_copy(V4.at[s, bk], vblk)
        pltpu.sync_copy(E4.at[s, bk], eblk)
        gather_fire(0, rows2.at[0])
        gather_fire(1, rows2.at[1])

        @pl.loop(0, TM)
        def _(wi):
            b = wi % NBUF
            gather_wait(rows2.at[b])

            @pl.when(wi + 2 < TM)
            def _():
                @pl.when(wi >= 1)
                def _():
                    pltpu.make_async_copy(
                        rows2.at[(wi + 2) % NBUF],
                        scat_ref.at[eblk.at[wi]], sem_s).wait()

                gather_fire(wi + 2, rows2.at[(wi + 2) % NBUF])

            pltpu.async_copy(rows2.at[b], scat_ref.at[eblk.at[wi]], sem_s,
                             add=True)

        for _k in range(NBUF):
            pltpu.make_async_copy(
                rows2.at[_k], scat_ref.at[eblk.at[0]], sem_s).wait()


def _msgpass_body(F2, xwl, xwr, V4, E4, inv_e_hbm,
                  outl, outr, vblk, eblk, rows2, invebuf,
                  sem_g, sem_s, e_sh, v_sh):
    c = lax.axis_index("c")
    s = lax.axis_index("s")
    ne_blks = NEP // NS // SB
    nv_blks = NVP // NS // SB
    zrows = rows2.at[0, pl.ds(0, SB), :]
    nref = rows2.at[0]

    pltpu.sync_copy(inv_e_hbm.at[pl.ds(s * (NEP // NS), NEP // NS)], invebuf)
    _zero_2d(nref, SB, F2)
    for k in range(ne_blks):
        pltpu.sync_copy(zrows, e_sh.at[pl.ds((s * ne_blks + k) * SB, SB), :])
    for k in range(nv_blks):
        pltpu.sync_copy(zrows, v_sh.at[pl.ds((s * nv_blks + k) * SB, SB), :])
    plsc.subcore_barrier()

    def fire1(wi, dst):
        @pl.when(c == 0)
        def _():
            pltpu.async_copy(xwl.at[vblk.at[wi]], dst, sem_g)

        @pl.when(c == 1)
        def _():
            pltpu.async_copy(xwr.at[vblk.at[wi]], dst, sem_g)

    def wait1(dst):
        pltpu.make_async_copy(xwl.at[vblk.at[0]], dst, sem_g).wait()

    _pipelined_stage(c, s, V4, E4, vblk, eblk, rows2, sem_g, sem_s,
                     fire1, wait1, e_sh)
    plsc.subcore_barrier()

    for k in range(ne_blks):
        base = (s * ne_blks + k) * SB
        lbase = k * SB
        pltpu.sync_copy(e_sh.at[pl.ds(base, SB), :], zrows)

        @pl.loop(0, SB // 16)
        def _(g):
            ivgrp = invebuf[pl.ds(lbase + g * 16, 16)]
            for r16 in range(16):
                iv = jnp.broadcast_to(ivgrp[r16], (16,))
                for j in range(F2 // 16):
                    r = g * 16 + r16
                    nref[r, pl.ds(j * 16, 16)] = nref[r, pl.ds(j * 16, 16)] * iv

        pltpu.sync_copy(zrows, e_sh.at[pl.ds(base, SB), :])

    plsc.subcore_barrier()

    def fire2(wi, dst):
        pltpu.async_copy(e_sh.at[eblk.at[wi]], dst, sem_g)

    def wait2(dst):
        pltpu.make_async_copy(e_sh.at[eblk.at[0]], dst, sem_g).wait()

    _pipelined_stage(c, s, E4, V4, eblk, vblk, rows2, sem_g, sem_s,
                     fire2, wait2, v_sh)
    plsc.subcore_barrier()

    nv_t = NVP // NS

    @pl.when(c == 0)
    def _():
        pltpu.sync_copy(v_sh.at[pl.ds(s * nv_t, nv_t), :],
                        outl.at[pl.ds(s * nv_t, nv_t), :])

    @pl.when(c == 1)
    def _():
        pltpu.sync_copy(v_sh.at[pl.ds(s * nv_t, nv_t), :],
                        outr.at[pl.ds(s * nv_t, nv_t), :])


@functools.cache
def _msgpass_kernel(F2):
    return pl.kernel(
        functools.partial(_msgpass_body, F2),
        out_type=(jax.ShapeDtypeStruct((NVP, F2), _f32),
                  jax.ShapeDtypeStruct((NVP, F2), _f32)),
        mesh=_MESH,
        scratch_types=[
            pltpu.VMEM((TM, WM), _i32),
            pltpu.VMEM((TM, WM), _i32),
            pltpu.VMEM((NBUF, WM, F2), _f32),
            pltpu.VMEM((NEP // NS,), _f32),
            pltpu.SemaphoreType.DMA,
            pltpu.SemaphoreType.DMA,
            pltpu.VMEM_SHARED((NEP, F2), _f32),
            pltpu.VMEM_SHARED((NVP, F2), _f32),
        ],
    )


_RBLK = 1000


def _matmul2_body(split, relu, wa_ref, wb_ref, b_ref, xa_ref, xb_ref, d_ref,
                  ol_ref, or_ref):
    xa, xb = xa_ref[...], xb_ref[...]
    if relu:
        xa, xb = jnp.maximum(xa, 0.0), jnp.maximum(xb, 0.0)
    acc = jnp.dot(xa, wa_ref[...], preferred_element_type=_f32)
    acc += jnp.dot(xb, wb_ref[...], preferred_element_type=_f32)
    acc = acc * d_ref[...] + b_ref[...]
    ol_ref[...] = acc[:, :split]
    or_ref[...] = acc[:, split:]


def _matmul2(xa, xb, w, b, d=None, relu=False):
    if d is None:
        d = jnp.ones((N, 1), _f32)
    fin_a = xa.shape[1]
    fout = w.shape[1]
    split = fout // 2
    grid = (N // _RBLK,)
    return pl.pallas_call(
        functools.partial(_matmul2_body, split, relu),
        grid=grid,
        in_specs=[
            pl.BlockSpec((fin_a, fout), lambda i: (0, 0)),
            pl.BlockSpec((xb.shape[1], fout), lambda i: (0, 0)),
            pl.BlockSpec((1, fout), lambda i: (0, 0)),
            pl.BlockSpec((_RBLK, fin_a), lambda i: (i, 0)),
            pl.BlockSpec((_RBLK, xb.shape[1]), lambda i: (i, 0)),
            pl.BlockSpec((_RBLK, 1), lambda i: (i, 0)),
        ],
        out_specs=[
            pl.BlockSpec((_RBLK, split), lambda i: (i, 0)),
            pl.BlockSpec((_RBLK, split), lambda i: (i, 0)),
        ],
        out_shape=[
            jax.ShapeDtypeStruct((N, split), _f32),
            jax.ShapeDtypeStruct((N, split), _f32),
        ],
    )(w[:fin_a], w[fin_a:], b.reshape(1, -1), xa, xb, d)


def _logsoftmax_body(xl_ref, xr_ref, d_ref, o_ref):
    x = jnp.concatenate([xl_ref[...], xr_ref[...]], axis=1) * d_ref[...]
    mask = lax.broadcasted_iota(_i32, x.shape, 1) < NCLASS
    neg = jnp.full_like(x, -jnp.inf)
    z = jnp.where(mask, x, neg)
    m = jnp.max(z, axis=1, keepdims=True)
    ex = jnp.where(mask, jnp.exp(x - m), 0.0)
    ssum = jnp.sum(ex, axis=1, keepdims=True)
    o_ref[...] = x - m - jnp.log(ssum)


def _logsoftmax(xl, xr, d):
    half = xl.shape[1]
    grid = (N // _RBLK,)
    return pl.pallas_call(
        _logsoftmax_body,
        grid=grid,
        in_specs=[
            pl.BlockSpec((_RBLK, half), lambda i: (i, 0)),
            pl.BlockSpec((_RBLK, half), lambda i: (i, 0)),
            pl.BlockSpec((_RBLK, 1), lambda i: (i, 0)),
        ],
        out_specs=pl.BlockSpec((_RBLK, 2 * half), lambda i: (i, 0)),
        out_shape=jax.ShapeDtypeStruct((N, 2 * half), _f32),
    )(xl, xr, d)


def kernel(X, V, E, W1, b1, W2, b2, W3, b3):
    V3 = V.astype(_i32).reshape(NS, NIB, IB, W)
    E3 = E.astype(_i32).reshape(NS, NIB, IB, W)
    V4 = V.astype(_i32).reshape(NS, NBLKM, TM, WM)
    E4 = E.astype(_i32).reshape(NS, NBLKM, TM, WM)
    inv_e, inv_v = _counts_kernel(V3, E3)
    d = inv_v[:N].reshape(N, 1)

    xl, xr = _matmul2(X[:, :NFEAT // 2], X[:, NFEAT // 2:], W1, b1)
    hl, hr = _msgpass_kernel(HID // 2)(xl, xr, V4, E4, inv_e)

    xl, xr = _matmul2(hl[:N], hr[:N], W2, b2, d=d, relu=True)
    hl, hr = _msgpass_kernel(HID // 2)(xl, xr, V4, E4, inv_e)

    w3p = jnp.pad(W3, ((0, 0), (0, NCLASS_PAD - NCLASS)))
    b3p = jnp.pad(b3, (0, NCLASS_PAD - NCLASS))
    xl, xr = _matmul2(hl[:N], hr[:N], w3p, b3p, d=d, relu=True)
    fl, fr = _msgpass_kernel(NCLASS_PAD // 2)(xl, xr, V4, E4, inv_e)

    out = _logsoftmax(fl[:N], fr[:N], d)
    return out[:, :NCLASS]


# speedup vs baseline: 1.4720x; 1.0389x over previous
import functools

import jax
import jax.numpy as jnp
from jax import lax
from jax.experimental import pallas as pl
from jax.experimental.pallas import tpu as pltpu
from jax.experimental.pallas import tpu_sc as plsc

N = 10000
NNZ = 160000
NE = 2000
NFEAT = 256
HID = 256
NCLASS = 40
NCLASS_PAD = 256

NS = 16
NC = 2
W = 80
PER_TILE = NNZ // NS
NWIN = PER_TILE // W
IB = 25
NIB = NWIN // IB
NEP = 2048
NVP = 10240
SB = 32

WM = 40
TM = 25
NBLKM = PER_TILE // (WM * TM)
NBUF = 4

_MESH = plsc.VectorSubcoreMesh(core_axis_name="c", subcore_axis_name="s")
_f32 = jnp.float32
_i32 = jnp.int32


def _zero_2d(buf, rows, cols):
    z = jnp.zeros((16,), _f32)

    @pl.loop(0, rows)
    def _(r):
        for j in range(cols // 16):
            buf[r, pl.ds(j * 16, 16)] = z


def _counts_body(V4, E4, inv_e_hbm, inv_v_hbm, vblk, eblk, ones, cbuf,
                 cnt_e_sh, cnt_v_sh):
    c = lax.axis_index("c")
    s = lax.axis_index("s")
    ne_t = NEP // NS
    nv_t = NVP // NS

    for j in range(W // 16):
        ones[pl.ds(j * 16, 16)] = jnp.ones((16,), _f32)
    for j in range(nv_t // 16):
        cbuf[pl.ds(j * 16, 16)] = jnp.zeros((16,), _f32)
    pltpu.sync_copy(cbuf, cnt_v_sh.at[pl.ds(s * nv_t, nv_t)])
    pltpu.sync_copy(cbuf.at[pl.ds(0, ne_t)], cnt_e_sh.at[pl.ds(s * ne_t, ne_t)])
    plsc.subcore_barrier()

    @pl.loop(0, NIB)
    def _(bk):
        pltpu.sync_copy(V4.at[s, bk], vblk)
        pltpu.sync_copy(E4.at[s, bk], eblk)

        @pl.loop(0, IB)
        def _(wi):
            pltpu.sync_copy(ones, cnt_e_sh.at[eblk.at[wi]], add=True)
            pltpu.sync_copy(ones, cnt_v_sh.at[vblk.at[wi]], add=True)

    plsc.subcore_barrier()

    pltpu.sync_copy(cnt_v_sh.at[pl.ds(s * nv_t, nv_t)], cbuf)

    @pl.loop(0, nv_t // 16)
    def _(j):
        x = cbuf[pl.ds(j * 16, 16)]
        cbuf[pl.ds(j * 16, 16)] = 1.0 / jnp.maximum(x, 1.0)

    @pl.when(c == 0)
    def _():
        pltpu.sync_copy(cbuf, inv_v_hbm.at[pl.ds(s * nv_t, nv_t)])

    pltpu.sync_copy(cnt_e_sh.at[pl.ds(s * ne_t, ne_t)], cbuf.at[pl.ds(0, ne_t)])

    @pl.loop(0, ne_t // 16)
    def _(j):
        x = cbuf[pl.ds(j * 16, 16)]
        cbuf[pl.ds(j * 16, 16)] = 1.0 / jnp.maximum(x, 1.0)

    @pl.when(c == 0)
    def _():
        pltpu.sync_copy(cbuf.at[pl.ds(0, ne_t)], inv_e_hbm.at[pl.ds(s * ne_t, ne_t)])


_counts_kernel = pl.kernel(
    _counts_body,
    out_type=(jax.ShapeDtypeStruct((NEP,), _f32),
              jax.ShapeDtypeStruct((NVP,), _f32)),
    mesh=_MESH,
    scratch_types=[
        pltpu.VMEM((IB, W), _i32),
        pltpu.VMEM((IB, W), _i32),
        pltpu.VMEM((W,), _f32),
        pltpu.VMEM((NVP // NS,), _f32),
        pltpu.VMEM_SHARED((NEP,), _f32),
        pltpu.VMEM_SHARED((NVP,), _f32),
    ],
)


def _pipelined_stage(c, s, V4, E4, vblk, eblk, rows2, sem_g, sem_s,
                     gather_fire, gather_wait, scat_ref):
    @pl.loop(0, NBLKM)
    def _(bk):
        pltpu.sync_copy(V4.at[s, bk], vblk)
        pltpu.sync_copy(E4.at[s, bk], eblk)
        gather_fire(0, rows2.at[0])
        gather_fire(1, rows2.at[1])
        gather_fire(2, rows2.at[2])

        @pl.loop(0, TM)
        def _(wi):
            b = wi % NBUF
            gather_wait(rows2.at[b])

            @pl.when(wi + 3 < TM)
            def _():
                @pl.when(wi >= 1)
                def _():
                    pltpu.make_async_copy(
                        rows2.at[(wi + 3) % NBUF],
                        scat_ref.at[eblk.at[wi]], sem_s).wait()

                gather_fire(wi + 3, rows2.at[(wi + 3) % NBUF])

            pltpu.async_copy(rows2.at[b], scat_ref.at[eblk.at[wi]], sem_s,
                             add=True)

        for _k in range(NBUF):
            pltpu.make_async_copy(
                rows2.at[_k], scat_ref.at[eblk.at[0]], sem_s).wait()


def _msgpass_body(F2, xwl, xwr, V4, E4, inv_e_hbm,
                  outl, outr, vblk, eblk, rows2, invebuf,
                  sem_g, sem_s, e_sh, v_sh):
    c = lax.axis_index("c")
    s = lax.axis_index("s")
    ne_blks = NEP // NS // SB
    nv_blks = NVP // NS // SB
    zrows = rows2.at[0, pl.ds(0, SB), :]
    nref = rows2.at[0]

    pltpu.sync_copy(inv_e_hbm.at[pl.ds(s * (NEP // NS), NEP // NS)], invebuf)
    _zero_2d(nref, SB, F2)
    for k in range(ne_blks):
        pltpu.sync_copy(zrows, e_sh.at[pl.ds((s * ne_blks + k) * SB, SB), :])
    for k in range(nv_blks):
        pltpu.sync_copy(zrows, v_sh.at[pl.ds((s * nv_blks + k) * SB, SB), :])
    plsc.subcore_barrier()

    def fire1(wi, dst):
        @pl.when(c == 0)
        def _():
            pltpu.async_copy(xwl.at[vblk.at[wi]], dst, sem_g)

        @pl.when(c == 1)
        def _():
            pltpu.async_copy(xwr.at[vblk.at[wi]], dst, sem_g)

    def wait1(dst):
        pltpu.make_async_copy(xwl.at[vblk.at[0]], dst, sem_g).wait()

    _pipelined_stage(c, s, V4, E4, vblk, eblk, rows2, sem_g, sem_s,
                     fire1, wait1, e_sh)
    plsc.subcore_barrier()

    for k in range(ne_blks):
        base = (s * ne_blks + k) * SB
        lbase = k * SB
        pltpu.sync_copy(e_sh.at[pl.ds(base, SB), :], zrows)

        @pl.loop(0, SB // 16)
        def _(g):
            ivgrp = invebuf[pl.ds(lbase + g * 16, 16)]
            for r16 in range(16):
                iv = jnp.broadcast_to(ivgrp[r16], (16,))
                for j in range(F2 // 16):
                    r = g * 16 + r16
                    nref[r, pl.ds(j * 16, 16)] = nref[r, pl.ds(j * 16, 16)] * iv

        pltpu.sync_copy(zrows, e_sh.at[pl.ds(base, SB), :])

    plsc.subcore_barrier()

    def fire2(wi, dst):
        pltpu.async_copy(e_sh.at[eblk.at[wi]], dst, sem_g)

    def wait2(dst):
        pltpu.make_async_copy(e_sh.at[eblk.at[0]], dst, sem_g).wait()

    _pipelined_stage(c, s, E4, V4, eblk, vblk, rows2, sem_g, sem_s,
                     fire2, wait2, v_sh)
    plsc.subcore_barrier()

    nv_t = NVP // NS

    @pl.when(c == 0)
    def _():
        pltpu.sync_copy(v_sh.at[pl.ds(s * nv_t, nv_t), :],
                        outl.at[pl.ds(s * nv_t, nv_t), :])

    @pl.when(c == 1)
    def _():
        pltpu.sync_copy(v_sh.at[pl.ds(s * nv_t, nv_t), :],
                        outr.at[pl.ds(s * nv_t, nv_t), :])


@functools.cache
def _msgpass_kernel(F2):
    return pl.kernel(
        functools.partial(_msgpass_body, F2),
        out_type=(jax.ShapeDtypeStruct((NVP, F2), _f32),
                  jax.ShapeDtypeStruct((NVP, F2), _f32)),
        mesh=_MESH,
        scratch_types=[
            pltpu.VMEM((TM, WM), _i32),
            pltpu.VMEM((TM, WM), _i32),
            pltpu.VMEM((NBUF, WM, F2), _f32),
            pltpu.VMEM((NEP // NS,), _f32),
            pltpu.SemaphoreType.DMA,
            pltpu.SemaphoreType.DMA,
            pltpu.VMEM_SHARED((NEP, F2), _f32),
            pltpu.VMEM_SHARED((NVP, F2), _f32),
        ],
    )


_RBLK = 1000


def _matmul2_body(split, relu, wa_ref, wb_ref, b_ref, xa_ref, xb_ref, d_ref,
                  ol_ref, or_ref):
    xa, xb = xa_ref[...], xb_ref[...]
    if relu:
        xa, xb = jnp.maximum(xa, 0.0), jnp.maximum(xb, 0.0)
    acc = jnp.dot(xa, wa_ref[...], preferred_element_type=_f32)
    acc += jnp.dot(xb, wb_ref[...], preferred_element_type=_f32)
    acc = acc * d_ref[...] + b_ref[...]
    ol_ref[...] = acc[:, :split]
    or_ref[...] = acc[:, split:]


def _matmul2(xa, xb, w, b, d=None, relu=False):
    if d is None:
        d = jnp.ones((N, 1), _f32)
    fin_a = xa.shape[1]
    fout = w.shape[1]
    split = fout // 2
    grid = (N // _RBLK,)
    return pl.pallas_call(
        functools.partial(_matmul2_body, split, relu),
        grid=grid,
        in_specs=[
            pl.BlockSpec((fin_a, fout), lambda i: (0, 0)),
            pl.BlockSpec((xb.shape[1], fout), lambda i: (0, 0)),
            pl.BlockSpec((1, fout), lambda i: (0, 0)),
            pl.BlockSpec((_RBLK, fin_a), lambda i: (i, 0)),
            pl.BlockSpec((_RBLK, xb.shape[1]), lambda i: (i, 0)),
            pl.BlockSpec((_RBLK, 1), lambda i: (i, 0)),
        ],
        out_specs=[
            pl.BlockSpec((_RBLK, split), lambda i: (i, 0)),
            pl.BlockSpec((_RBLK, split), lambda i: (i, 0)),
        ],
        out_shape=[
            jax.ShapeDtypeStruct((N, split), _f32),
            jax.ShapeDtypeStruct((N, split), _f32),
        ],
    )(w[:fin_a], w[fin_a:], b.reshape(1, -1), xa, xb, d)


def _logsoftmax_body(xl_ref, xr_ref, d_ref, o_ref):
    x = jnp.concatenate([xl_ref[...], xr_ref[...]], axis=1) * d_ref[...]
    mask = lax.broadcasted_iota(_i32, x.shape, 1) < NCLASS
    neg = jnp.full_like(x, -jnp.inf)
    z = jnp.where(mask, x, neg)
    m = jnp.max(z, axis=1, keepdims=True)
    ex = jnp.where(mask, jnp.exp(x - m), 0.0)
    ssum = jnp.sum(ex, axis=1, keepdims=True)
    o_ref[...] = x - m - jnp.log(ssum)


def _logsoftmax(xl, xr, d):
    half = xl.shape[1]
    grid = (N // _RBLK,)
    return pl.pallas_call(
        _logsoftmax_body,
        grid=grid,
        in_specs=[
            pl.BlockSpec((_RBLK, half), lambda i: (i, 0)),
            pl.BlockSpec((_RBLK, half), lambda i: (i, 0)),
            pl.BlockSpec((_RBLK, 1), lambda i: (i, 0)),
        ],
        out_specs=pl.BlockSpec((_RBLK, 2 * half), lambda i: (i, 0)),
        out_shape=jax.ShapeDtypeStruct((N, 2 * half), _f32),
    )(xl, xr, d)


def kernel(X, V, E, W1, b1, W2, b2, W3, b3):
    V3 = V.astype(_i32).reshape(NS, NIB, IB, W)
    E3 = E.astype(_i32).reshape(NS, NIB, IB, W)
    V4 = V.astype(_i32).reshape(NS, NBLKM, TM, WM)
    E4 = E.astype(_i32).reshape(NS, NBLKM, TM, WM)
    inv_e, inv_v = _counts_kernel(V3, E3)
    d = inv_v[:N].reshape(N, 1)

    xl, xr = _matmul2(X[:, :NFEAT // 2], X[:, NFEAT // 2:], W1, b1)
    hl, hr = _msgpass_kernel(HID // 2)(xl, xr, V4, E4, inv_e)

    xl, xr = _matmul2(hl[:N], hr[:N], W2, b2, d=d, relu=True)
    hl, hr = _msgpass_kernel(HID // 2)(xl, xr, V4, E4, inv_e)

    w3p = jnp.pad(W3, ((0, 0), (0, NCLASS_PAD - NCLASS)))
    b3p = jnp.pad(b3, (0, NCLASS_PAD - NCLASS))
    xl, xr = _matmul2(hl[:N], hr[:N], w3p, b3p, d=d, relu=True)
    fl, fr = _msgpass_kernel(NCLASS_PAD // 2)(xl, xr, V4, E4, inv_e)

    out = _logsoftmax(fl[:N], fr[:N], d)
    return out[:, :NCLASS]


# speedup vs baseline: 1.5389x; 1.0454x over previous
import functools

import jax
import jax.numpy as jnp
from jax import lax
from jax.experimental import pallas as pl
from jax.experimental.pallas import tpu as pltpu
from jax.experimental.pallas import tpu_sc as plsc

N = 10000
NNZ = 160000
NE = 2000
NFEAT = 256
HID = 256
NCLASS = 40
NCLASS_PAD = 256

NS = 16
NC = 2
W = 80
PER_TILE = NNZ // NS
NWIN = PER_TILE // W
IB = 25
NIB = NWIN // IB
NEP = 2048
NVP = 10240
SB = 32

WM = 40
TM = 25
NBLKM = PER_TILE // (WM * TM)
NBUF = 4

_MESH = plsc.VectorSubcoreMesh(core_axis_name="c", subcore_axis_name="s")
_f32 = jnp.float32
_i32 = jnp.int32


def _zero_2d(buf, rows, cols):
    z = jnp.zeros((16,), _f32)

    @pl.loop(0, rows)
    def _(r):
        for j in range(cols // 16):
            buf[r, pl.ds(j * 16, 16)] = z


def _counts_body(V4, E4, inv_e_hbm, inv_v_hbm, vblk, ones, cbuf,
                 cnt_e_sh, cnt_v_sh):
    c = lax.axis_index("c")
    s = lax.axis_index("s")
    ne_t = NEP // NS
    nv_t = NVP // NS

    for j in range(W // 16):
        ones[pl.ds(j * 16, 16)] = jnp.ones((16,), _f32)
    for j in range(nv_t // 16):
        cbuf[pl.ds(j * 16, 16)] = jnp.zeros((16,), _f32)
    pltpu.sync_copy(cbuf, cnt_v_sh.at[pl.ds(s * nv_t, nv_t)])
    pltpu.sync_copy(cbuf.at[pl.ds(0, ne_t)], cnt_e_sh.at[pl.ds(s * ne_t, ne_t)])
    plsc.subcore_barrier()

    @pl.loop(0, NIB)
    def _(bk):
        @pl.when(c == 0)
        def _():
            pltpu.sync_copy(V4.at[s, bk], vblk)

        @pl.when(c == 1)
        def _():
            pltpu.sync_copy(E4.at[s, bk], vblk)

        @pl.loop(0, IB)
        def _(wi):
            @pl.when(c == 0)
            def _():
                pltpu.sync_copy(ones, cnt_v_sh.at[vblk.at[wi]], add=True)

            @pl.when(c == 1)
            def _():
                pltpu.sync_copy(ones, cnt_e_sh.at[vblk.at[wi]], add=True)

    plsc.subcore_barrier()

    @pl.when(c == 0)
    def _():
        pltpu.sync_copy(cnt_v_sh.at[pl.ds(s * nv_t, nv_t)], cbuf)

        @pl.loop(0, nv_t // 16)
        def _(j):
            x = cbuf[pl.ds(j * 16, 16)]
            cbuf[pl.ds(j * 16, 16)] = 1.0 / jnp.maximum(x, 1.0)

        pltpu.sync_copy(cbuf, inv_v_hbm.at[pl.ds(s * nv_t, nv_t)])

    @pl.when(c == 1)
    def _():
        pltpu.sync_copy(cnt_e_sh.at[pl.ds(s * ne_t, ne_t)], cbuf.at[pl.ds(0, ne_t)])

        @pl.loop(0, ne_t // 16)
        def _(j):
            x = cbuf[pl.ds(j * 16, 16)]
            cbuf[pl.ds(j * 16, 16)] = 1.0 / jnp.maximum(x, 1.0)

        pltpu.sync_copy(cbuf.at[pl.ds(0, ne_t)], inv_e_hbm.at[pl.ds(s * ne_t, ne_t)])


_counts_kernel = pl.kernel(
    _counts_body,
    out_type=(jax.ShapeDtypeStruct((NEP,), _f32),
              jax.ShapeDtypeStruct((NVP,), _f32)),
    mesh=_MESH,
    scratch_types=[
        pltpu.VMEM((IB, W), _i32),
        pltpu.VMEM((W,), _f32),
        pltpu.VMEM((NVP // NS,), _f32),
        pltpu.VMEM_SHARED((NEP,), _f32),
        pltpu.VMEM_SHARED((NVP,), _f32),
    ],
)


def _pipelined_stage(c, s, V4, E4, vblk, eblk, rows2, sem_g, sem_s,
                     gather_fire, gather_wait, scat_ref):
    @pl.loop(0, NBLKM)
    def _(bk):
        pltpu.sync_copy(V4.at[s, bk], vblk)
        pltpu.sync_copy(E4.at[s, bk], eblk)
        gather_fire(0, rows2.at[0])
        gather_fire(1, rows2.at[1])
        gather_fire(2, rows2.at[2])

        @pl.loop(0, TM)
        def _(wi):
            b = wi % NBUF
            gather_wait(rows2.at[b])

            @pl.when(wi + 3 < TM)
            def _():
                @pl.when(wi >= 1)
                def _():
                    pltpu.make_async_copy(
                        rows2.at[(wi + 3) % NBUF],
                        scat_ref.at[eblk.at[wi]], sem_s).wait()

                gather_fire(wi + 3, rows2.at[(wi + 3) % NBUF])

            pltpu.async_copy(rows2.at[b], scat_ref.at[eblk.at[wi]], sem_s,
                             add=True)

        for _k in range(NBUF):
            pltpu.make_async_copy(
                rows2.at[_k], scat_ref.at[eblk.at[0]], sem_s).wait()


def _msgpass_body(F2, xwl, xwr, V4, E4, inv_e_hbm,
                  outl, outr, vblk, eblk, rows2, invebuf,
                  sem_g, sem_s, e_sh, v_sh):
    c = lax.axis_index("c")
    s = lax.axis_index("s")
    ne_blks = NEP // NS // SB
    nv_blks = NVP // NS // SB
    zrows = rows2.at[0, pl.ds(0, SB), :]
    nref = rows2.at[0]

    pltpu.sync_copy(inv_e_hbm.at[pl.ds(s * (NEP // NS), NEP // NS)], invebuf)
    _zero_2d(nref, SB, F2)
    for k in range(ne_blks):
        pltpu.sync_copy(zrows, e_sh.at[pl.ds((s * ne_blks + k) * SB, SB), :])
    for k in range(nv_blks):
        pltpu.sync_copy(zrows, v_sh.at[pl.ds((s * nv_blks + k) * SB, SB), :])
    plsc.subcore_barrier()

    def fire1(wi, dst):
        @pl.when(c == 0)
        def _():
            pltpu.async_copy(xwl.at[vblk.at[wi]], dst, sem_g)

        @pl.when(c == 1)
        def _():
            pltpu.async_copy(xwr.at[vblk.at[wi]], dst, sem_g)

    def wait1(dst):
        pltpu.make_async_copy(xwl.at[vblk.at[0]], dst, sem_g).wait()

    _pipelined_stage(c, s, V4, E4, vblk, eblk, rows2, sem_g, sem_s,
                     fire1, wait1, e_sh)
    plsc.subcore_barrier()

    for k in range(ne_blks):
        base = (s * ne_blks + k) * SB
        lbase = k * SB
        pltpu.sync_copy(e_sh.at[pl.ds(base, SB), :], zrows)

        @pl.loop(0, SB // 16)
        def _(g):
            ivgrp = invebuf[pl.ds(lbase + g * 16, 16)]
            for r16 in range(16):
                iv = jnp.broadcast_to(ivgrp[r16], (16,))
                for j in range(F2 // 16):
                    r = g * 16 + r16
                    nref[r, pl.ds(j * 16, 16)] = nref[r, pl.ds(j * 16, 16)] * iv

        pltpu.sync_copy(zrows, e_sh.at[pl.ds(base, SB), :])

    plsc.subcore_barrier()

    def fire2(wi, dst):
        pltpu.async_copy(e_sh.at[eblk.at[wi]], dst, sem_g)

    def wait2(dst):
        pltpu.make_async_copy(e_sh.at[eblk.at[0]], dst, sem_g).wait()

    _pipelined_stage(c, s, E4, V4, eblk, vblk, rows2, sem_g, sem_s,
                     fire2, wait2, v_sh)
    plsc.subcore_barrier()

    nv_t = NVP // NS

    @pl.when(c == 0)
    def _():
        pltpu.sync_copy(v_sh.at[pl.ds(s * nv_t, nv_t), :],
                        outl.at[pl.ds(s * nv_t, nv_t), :])

    @pl.when(c == 1)
    def _():
        pltpu.sync_copy(v_sh.at[pl.ds(s * nv_t, nv_t), :],
                        outr.at[pl.ds(s * nv_t, nv_t), :])


@functools.cache
def _msgpass_kernel(F2):
    return pl.kernel(
        functools.partial(_msgpass_body, F2),
        out_type=(jax.ShapeDtypeStruct((NVP, F2), _f32),
                  jax.ShapeDtypeStruct((NVP, F2), _f32)),
        mesh=_MESH,
        scratch_types=[
            pltpu.VMEM((TM, WM), _i32),
            pltpu.VMEM((TM, WM), _i32),
            pltpu.VMEM((NBUF, WM, F2), _f32),
            pltpu.VMEM((NEP // NS,), _f32),
            pltpu.SemaphoreType.DMA,
            pltpu.SemaphoreType.DMA,
            pltpu.VMEM_SHARED((NEP, F2), _f32),
            pltpu.VMEM_SHARED((NVP, F2), _f32),
        ],
    )


_RBLK = 1000


def _matmul2_body(split, relu, wa_ref, wb_ref, b_ref, xa_ref, xb_ref, d_ref,
                  ol_ref, or_ref):
    xa, xb = xa_ref[...], xb_ref[...]
    if relu:
        xa, xb = jnp.maximum(xa, 0.0), jnp.maximum(xb, 0.0)
    acc = jnp.dot(xa, wa_ref[...], preferred_element_type=_f32)
    acc += jnp.dot(xb, wb_ref[...], preferred_element_type=_f32)
    acc = acc * d_ref[...] + b_ref[...]
    ol_ref[...] = acc[:, :split]
    or_ref[...] = acc[:, split:]


def _matmul2(xa, xb, w, b, d=None, relu=False, ca=0, cb=0):
    if d is None:
        d = jnp.ones((N, 1), _f32)
    fin_a = HID // 2
    fout = w.shape[1]
    split = fout // 2
    grid = (N // _RBLK,)
    return pl.pallas_call(
        functools.partial(_matmul2_body, split, relu),
        grid=grid,
        in_specs=[
            pl.BlockSpec((fin_a, fout), lambda i: (0, 0)),
            pl.BlockSpec((fin_a, fout), lambda i: (0, 0)),
            pl.BlockSpec((1, fout), lambda i: (0, 0)),
            pl.BlockSpec((_RBLK, fin_a), lambda i: (i, ca)),
            pl.BlockSpec((_RBLK, fin_a), lambda i: (i, cb)),
            pl.BlockSpec((_RBLK, 1), lambda i: (i, 0)),
        ],
        out_specs=[
            pl.BlockSpec((_RBLK, split), lambda i: (i, 0)),
            pl.BlockSpec((_RBLK, split), lambda i: (i, 0)),
        ],
        out_shape=[
            jax.ShapeDtypeStruct((N, split), _f32),
            jax.ShapeDtypeStruct((N, split), _f32),
        ],
    )(w[:fin_a], w[fin_a:], b.reshape(1, -1), xa, xb, d)


def _logsoftmax_body(xl_ref, xr_ref, d_ref, o_ref):
    x = jnp.concatenate([xl_ref[...], xr_ref[...]], axis=1) * d_ref[...]
    mask = lax.broadcasted_iota(_i32, x.shape, 1) < NCLASS
    neg = jnp.full_like(x, -jnp.inf)
    z = jnp.where(mask, x, neg)
    m = jnp.max(z, axis=1, keepdims=True)
    ex = jnp.where(mask, jnp.exp(x - m), 0.0)
    ssum = jnp.sum(ex, axis=1, keepdims=True)
    o_ref[...] = x - m - jnp.log(ssum)


def _logsoftmax(xl, xr, d):
    half = xl.shape[1]
    grid = (N // _RBLK,)
    return pl.pallas_call(
        _logsoftmax_body,
        grid=grid,
        in_specs=[
            pl.BlockSpec((_RBLK, half), lambda i: (i, 0)),
            pl.BlockSpec((_RBLK, half), lambda i: (i, 0)),
            pl.BlockSpec((_RBLK, 1), lambda i: (i, 0)),
        ],
        out_specs=pl.BlockSpec((_RBLK, 2 * half), lambda i: (i, 0)),
        out_shape=jax.ShapeDtypeStruct((N, 2 * half), _f32),
    )(xl, xr, d)


def kernel(X, V, E, W1, b1, W2, b2, W3, b3):
    V3 = V.astype(_i32).reshape(NS, NIB, IB, W)
    E3 = E.astype(_i32).reshape(NS, NIB, IB, W)
    V4 = V.astype(_i32).reshape(NS, NBLKM, TM, WM)
    E4 = E.astype(_i32).reshape(NS, NBLKM, TM, WM)
    inv_e, inv_v = _counts_kernel(V3, E3)
    d = inv_v[:N].reshape(N, 1)

    xl, xr = _matmul2(X, X, W1, b1, ca=0, cb=1)
    hl, hr = _msgpass_kernel(HID // 2)(xl, xr, V4, E4, inv_e)

    xl, xr = _matmul2(hl, hr, W2, b2, d=d, relu=True)
    hl, hr = _msgpass_kernel(HID // 2)(xl, xr, V4, E4, inv_e)

    w3p = jnp.pad(W3, ((0, 0), (0, NCLASS_PAD - NCLASS)))
    b3p = jnp.pad(b3, (0, NCLASS_PAD - NCLASS))
    xl, xr = _matmul2(hl, hr, w3p, b3p, d=d, relu=True)
    fl, fr = _msgpass_kernel(NCLASS_PAD // 2)(xl, xr, V4, E4, inv_e)

    out = _logsoftmax(fl, fr, d)
    return out[:, :NCLASS]
